# Initial kernel scaffold; baseline (speedup 1.0000x reference)
import jax
import jax.numpy as jnp
from jax.experimental import pallas as pl


def kernel(x, weights, top_weights, top_experts, w1, w2, bias):
    raise NotImplementedError("write your pallas kernel here")


# speedup vs baseline: 5.2314x; 5.2314x over previous
import functools

import jax
import jax.numpy as jnp
from jax.experimental import pallas as pl
from jax.experimental.pallas import tpu as pltpu
from jax.experimental.pallas import tpu_sc as plsc

BT = 128
SC_W = 128
COL_SPLIT = 4


def _sc_mesh():
    return plsc.VectorSubcoreMesh(core_axis_name="c", subcore_axis_name="s")


def _sc_gather(x, tw16, sidx4, sidx):
    t, d = x.shape
    ds = d // COL_SPLIT
    tx = t * COL_SPLIT
    xv = x.reshape(tx, ds)

    @pl.kernel(
        out_type=(
            jax.ShapeDtypeStruct((tx, ds), x.dtype),
            jax.ShapeDtypeStruct(tw16.shape, tw16.dtype),
        ),
        mesh=_sc_mesh(),
    )
    def k(x_hbm, tw_hbm, i4_hbm, i_hbm, ox_hbm, otw_hbm):
        def bodyx(i_vmem, ox_vmem):
            pltpu.sync_copy(x_hbm.at[i_vmem.at[0]], ox_vmem)

        pltpu.emit_pipeline(
            bodyx,
            grid=(tx // SC_W,),
            in_specs=[pl.BlockSpec((1, SC_W), lambda i: (0, i))],
            out_specs=[pl.BlockSpec((SC_W, ds), lambda i: (i, 0))],
            core_axis_name=("c", "s"),
            dimension_semantics=(pltpu.PARALLEL,),
        )(i4_hbm, ox_hbm)

        def bodyt(i_vmem, otw_vmem):
            pltpu.sync_copy(tw_hbm.at[i_vmem.at[0]], otw_vmem)

        pltpu.emit_pipeline(
            bodyt,
            grid=(t // SC_W,),
            in_specs=[pl.BlockSpec((1, SC_W), lambda i: (0, i))],
            out_specs=[pl.BlockSpec((SC_W, tw16.shape[1]), lambda i: (i, 0))],
            core_axis_name=("c", "s"),
            dimension_semantics=(pltpu.PARALLEL,),
        )(i_hbm, otw_hbm)

    xs, tws = k(xv, tw16, sidx4.reshape(1, tx), sidx.reshape(1, t))
    return xs.reshape(t, d), tws


def _sc_scatter(y, sidx4):
    t, d = y.shape
    ds = d // COL_SPLIT
    tx = t * COL_SPLIT
    yv = y.reshape(tx, ds)

    @pl.kernel(out_type=jax.ShapeDtypeStruct((tx, ds), y.dtype), mesh=_sc_mesh())
    def k(y_hbm, i_hbm, o_hbm):
        def body(y_vmem, i_vmem):
            pltpu.sync_copy(y_vmem, o_hbm.at[i_vmem.at[0]])

        pltpu.emit_pipeline(
            body,
            grid=(tx // SC_W,),
            in_specs=[
                pl.BlockSpec((SC_W, ds), lambda i: (i, 0)),
                pl.BlockSpec((1, SC_W), lambda i: (0, i)),
            ],
            out_specs=[],
            core_axis_name=("c", "s"),
            dimension_semantics=(pltpu.PARALLEL,),
        )(y_hbm, i_hbm)

    return k(yv, sidx4.reshape(1, tx)).reshape(t, d)


def _first_visit(meta_ref, i):
    rb = meta_ref[0, i]
    prev = meta_ref[0, jnp.maximum(i - 1, 0)]
    return jnp.logical_or(i == 0, rb != prev)


def _row_mask(meta_ref, i, bt):
    lo = meta_ref[2, i]
    hi = meta_ref[3, i]
    rows = jax.lax.broadcasted_iota(jnp.int32, (bt, 1), 0)
    return jnp.logical_and(rows >= lo, rows < hi)


def _pass1_body(meta_ref, x_ref, w1_ref, o_ref):
    i = pl.program_id(0)
    mask = _row_mask(meta_ref, i, o_ref.shape[0])
    xb = jnp.where(mask, x_ref[...], 0.0).astype(jnp.bfloat16)
    w = w1_ref[0].astype(jnp.bfloat16)
    h = jax.lax.dot_general(
        xb, w, (((1,), (1,)), ((), ())), preferred_element_type=jnp.float32
    )
    a = 0.5 * h * (1.0 + jax.lax.erf(h * 0.7071067811865476))

    @pl.when(_first_visit(meta_ref, i))
    def _():
        o_ref[...] = jnp.zeros_like(o_ref)

    o_ref[...] += a.astype(jnp.bfloat16)


def _pass2_body(meta_ref, a_ref, w2_ref, tw_ref, b_ref, o_ref):
    i = pl.program_id(0)
    mask = _row_mask(meta_ref, i, o_ref.shape[0])
    a = jnp.where(mask, a_ref[...], jnp.bfloat16(0))
    w = w2_ref[0].astype(jnp.bfloat16)
    y = jax.lax.dot_general(
        a, w, (((1,), (0,)), ((), ())), preferred_element_type=jnp.float32
    )

    @pl.when(_first_visit(meta_ref, i))
    def _():
        o_ref[...] = jnp.broadcast_to(b_ref[...], o_ref.shape)

    o_ref[...] += y * tw_ref[:, :1]


def _grouped_mlp(meta, x_sorted, tw_sorted, w1r, w2r, bias2d, nw):
    t, d = x_sorted.shape
    e, f, _ = w1r.shape

    act = pl.pallas_call(
        _pass1_body,
        grid_spec=pltpu.PrefetchScalarGridSpec(
            num_scalar_prefetch=1,
            grid=(nw,),
            in_specs=[
                pl.BlockSpec((BT, d), lambda i, m: (m[0, i], 0)),
                pl.BlockSpec((1, f, d), lambda i, m: (m[1, i], 0, 0)),
            ],
            out_specs=pl.BlockSpec((BT, f), lambda i, m: (m[0, i], 0)),
        ),
        out_shape=jax.ShapeDtypeStruct((t, f), jnp.bfloat16),
    )(meta, x_sorted, w1r)

    y_sorted = pl.pallas_call(
        _pass2_body,
        grid_spec=pltpu.PrefetchScalarGridSpec(
            num_scalar_prefetch=1,
            grid=(nw,),
            in_specs=[
                pl.BlockSpec((BT, f), lambda i, m: (m[0, i], 0)),
                pl.BlockSpec((1, f, d), lambda i, m: (m[1, i], 0, 0)),
                pl.BlockSpec((BT, 128), lambda i, m: (m[0, i], 0)),
                pl.BlockSpec((1, d), lambda i, m: (0, 0)),
            ],
            out_specs=pl.BlockSpec((BT, d), lambda i, m: (m[0, i], 0)),
        ),
        out_shape=jax.ShapeDtypeStruct((t, d), jnp.float32),
    )(meta, act, w2r, tw_sorted, bias2d)
    return y_sorted


def kernel(x, weights, top_weights, top_experts, w1, w2, bias):
    t, d = x.shape
    ew_f, _ = w1.shape
    e = weights.shape[1]
    f = ew_f // e
    nt = t // BT
    nw = nt + e - 1

    eidx = top_experts[:, 0].astype(jnp.int32)
    sidx = jnp.argsort(eidx).astype(jnp.int32)
    sorted_e = eidx[sidx]
    offsets = jnp.searchsorted(sorted_e, jnp.arange(e + 1, dtype=jnp.int32)).astype(
        jnp.int32
    )

    tile_bnds = jnp.arange(nt + 1, dtype=jnp.int32) * BT
    bnds = jnp.sort(jnp.concatenate([tile_bnds, offsets[1:e]]))
    seg_start = bnds[:nw]
    seg_end = bnds[1 : nw + 1]
    rb = jnp.clip(seg_start // BT, 0, nt - 1)
    lo = seg_start - rb * BT
    hi = seg_end - rb * BT
    expert = jnp.clip(
        jnp.searchsorted(offsets, seg_start, side="right").astype(jnp.int32) - 1,
        0,
        e - 1,
    )
    meta = jnp.stack([rb, expert, lo, hi]).astype(jnp.int32)

    sidx4 = (
        sidx[:, None] * COL_SPLIT + jnp.arange(COL_SPLIT, dtype=jnp.int32)[None, :]
    ).reshape(-1)
    tw16 = jnp.broadcast_to(top_weights[:, :1], (t, 128))
    x_sorted, tw_sorted = _sc_gather(x, tw16, sidx4, sidx)

    w1r = w1.reshape(e, f, d)
    w2r = w2.reshape(e, f, d)
    y_sorted = _grouped_mlp(meta, x_sorted, tw_sorted, w1r, w2r, bias.reshape(1, d), nw)

    return _sc_scatter(y_sorted, sidx4)


# speedup vs baseline: 5.8405x; 1.1164x over previous
import functools

import jax
import jax.numpy as jnp
from jax.experimental import pallas as pl
from jax.experimental.pallas import tpu as pltpu
from jax.experimental.pallas import tpu_sc as plsc

BT = 256
SC_W = 128
COL_SPLIT = 4


def _sc_mesh():
    return plsc.VectorSubcoreMesh(core_axis_name="c", subcore_axis_name="s")


def _sc_gather(x, tw16, sidx4, sidx):
    t, d = x.shape
    ds = d // COL_SPLIT
    tx = t * COL_SPLIT
    xv = x.reshape(tx, ds)

    @pl.kernel(
        out_type=(
            jax.ShapeDtypeStruct((tx, ds), x.dtype),
            jax.ShapeDtypeStruct(tw16.shape, tw16.dtype),
        ),
        mesh=_sc_mesh(),
    )
    def k(x_hbm, tw_hbm, i4_hbm, i_hbm, ox_hbm, otw_hbm):
        def bodyx(i_vmem, ox_vmem):
            pltpu.sync_copy(x_hbm.at[i_vmem.at[0]], ox_vmem)

        pltpu.emit_pipeline(
            bodyx,
            grid=(tx // SC_W,),
            in_specs=[pl.BlockSpec((1, SC_W), lambda i: (0, i))],
            out_specs=[pl.BlockSpec((SC_W, ds), lambda i: (i, 0))],
            core_axis_name=("c", "s"),
            dimension_semantics=(pltpu.PARALLEL,),
        )(i4_hbm, ox_hbm)

        def bodyt(i_vmem, otw_vmem):
            pltpu.sync_copy(tw_hbm.at[i_vmem.at[0]], otw_vmem)

        pltpu.emit_pipeline(
            bodyt,
            grid=(t // SC_W,),
            in_specs=[pl.BlockSpec((1, SC_W), lambda i: (0, i))],
            out_specs=[pl.BlockSpec((SC_W, tw16.shape[1]), lambda i: (i, 0))],
            core_axis_name=("c", "s"),
            dimension_semantics=(pltpu.PARALLEL,),
        )(i_hbm, otw_hbm)

    xs, tws = k(xv, tw16, sidx4.reshape(1, tx), sidx.reshape(1, t))
    return xs.reshape(t, d), tws


def _sc_scatter(y, sidx4):
    t, d = y.shape
    ds = d // COL_SPLIT
    tx = t * COL_SPLIT
    yv = y.reshape(tx, ds)

    @pl.kernel(out_type=jax.ShapeDtypeStruct((tx, ds), y.dtype), mesh=_sc_mesh())
    def k(y_hbm, i_hbm, o_hbm):
        def body(y_vmem, i_vmem):
            pltpu.sync_copy(y_vmem, o_hbm.at[i_vmem.at[0]])

        pltpu.emit_pipeline(
            body,
            grid=(tx // SC_W,),
            in_specs=[
                pl.BlockSpec((SC_W, ds), lambda i: (i, 0)),
                pl.BlockSpec((1, SC_W), lambda i: (0, i)),
            ],
            out_specs=[],
            core_axis_name=("c", "s"),
            dimension_semantics=(pltpu.PARALLEL,),
        )(y_hbm, i_hbm)

    return k(yv, sidx4.reshape(1, tx)).reshape(t, d)


def _first_visit(meta_ref, i):
    rb = meta_ref[0, i]
    prev = meta_ref[0, jnp.maximum(i - 1, 0)]
    return jnp.logical_or(i == 0, rb != prev)


def _row_mask(meta_ref, i, bt):
    lo = meta_ref[2, i]
    hi = meta_ref[3, i]
    rows = jax.lax.broadcasted_iota(jnp.int32, (bt, 1), 0)
    return jnp.logical_and(rows >= lo, rows < hi)


def _pass1_body(meta_ref, x_ref, w1_ref, o_ref):
    i = pl.program_id(0)
    mask = _row_mask(meta_ref, i, o_ref.shape[0])
    xb = jnp.where(mask, x_ref[...], 0.0).astype(jnp.bfloat16)
    w = w1_ref[0].astype(jnp.bfloat16)
    h = jax.lax.dot_general(
        xb, w, (((1,), (1,)), ((), ())), preferred_element_type=jnp.float32
    )
    a = 0.5 * h * (1.0 + jax.lax.erf(h * 0.7071067811865476))

    @pl.when(_first_visit(meta_ref, i))
    def _():
        o_ref[...] = jnp.zeros_like(o_ref)

    o_ref[...] += a.astype(jnp.bfloat16)


def _pass2_body(meta_ref, a_ref, w2_ref, tw_ref, b_ref, o_ref):
    i = pl.program_id(0)
    mask = _row_mask(meta_ref, i, o_ref.shape[0])
    a = jnp.where(mask, a_ref[...], jnp.bfloat16(0))
    w = w2_ref[0].astype(jnp.bfloat16)
    y = jax.lax.dot_general(
        a, w, (((1,), (0,)), ((), ())), preferred_element_type=jnp.float32
    )

    @pl.when(_first_visit(meta_ref, i))
    def _():
        o_ref[...] = jnp.broadcast_to(b_ref[...], o_ref.shape)

    o_ref[...] += y * tw_ref[:, :1]


def _grouped_mlp(meta, x_sorted, tw_sorted, w1r, w2r, bias2d, nw):
    t, d = x_sorted.shape
    e, f, _ = w1r.shape

    act = pl.pallas_call(
        _pass1_body,
        grid_spec=pltpu.PrefetchScalarGridSpec(
            num_scalar_prefetch=1,
            grid=(nw,),
            in_specs=[
                pl.BlockSpec((BT, d), lambda i, m: (m[0, i], 0)),
                pl.BlockSpec((1, f, d), lambda i, m: (m[1, i], 0, 0)),
            ],
            out_specs=pl.BlockSpec((BT, f), lambda i, m: (m[0, i], 0)),
        ),
        out_shape=jax.ShapeDtypeStruct((t, f), jnp.bfloat16),
    )(meta, x_sorted, w1r)

    y_sorted = pl.pallas_call(
        _pass2_body,
        grid_spec=pltpu.PrefetchScalarGridSpec(
            num_scalar_prefetch=1,
            grid=(nw,),
            in_specs=[
                pl.BlockSpec((BT, f), lambda i, m: (m[0, i], 0)),
                pl.BlockSpec((1, f, d), lambda i, m: (m[1, i], 0, 0)),
                pl.BlockSpec((BT, 128), lambda i, m: (m[0, i], 0)),
                pl.BlockSpec((1, d), lambda i, m: (0, 0)),
            ],
            out_specs=pl.BlockSpec((BT, d), lambda i, m: (m[0, i], 0)),
        ),
        out_shape=jax.ShapeDtypeStruct((t, d), jnp.float32),
    )(meta, act, w2r, tw_sorted, bias2d)
    return y_sorted


def kernel(x, weights, top_weights, top_experts, w1, w2, bias):
    t, d = x.shape
    ew_f, _ = w1.shape
    e = weights.shape[1]
    f = ew_f // e
    nt = t // BT
    nw = nt + e - 1

    eidx = top_experts[:, 0].astype(jnp.int32)
    sidx = jnp.argsort(eidx).astype(jnp.int32)
    sorted_e = eidx[sidx]
    offsets = jnp.searchsorted(sorted_e, jnp.arange(e + 1, dtype=jnp.int32)).astype(
        jnp.int32
    )

    tile_bnds = jnp.arange(nt + 1, dtype=jnp.int32) * BT
    bnds = jnp.sort(jnp.concatenate([tile_bnds, offsets[1:e]]))
    seg_start = bnds[:nw]
    seg_end = bnds[1 : nw + 1]
    rb = jnp.clip(seg_start // BT, 0, nt - 1)
    lo = seg_start - rb * BT
    hi = seg_end - rb * BT
    expert = jnp.clip(
        jnp.searchsorted(offsets, seg_start, side="right").astype(jnp.int32) - 1,
        0,
        e - 1,
    )
    meta = jnp.stack([rb, expert, lo, hi]).astype(jnp.int32)

    sidx4 = (
        sidx[:, None] * COL_SPLIT + jnp.arange(COL_SPLIT, dtype=jnp.int32)[None, :]
    ).reshape(-1)
    tw16 = jnp.broadcast_to(top_weights[:, :1], (t, 128))
    x_sorted, tw_sorted = _sc_gather(x, tw16, sidx4, sidx)

    w1r = w1.reshape(e, f, d)
    w2r = w2.reshape(e, f, d)
    y_sorted = _grouped_mlp(meta, x_sorted, tw_sorted, w1r, w2r, bias.reshape(1, d), nw)

    return _sc_scatter(y_sorted, sidx4)
